# baseline (speedup 1.0000x reference)
import functools

import jax
import jax.numpy as jnp
from jax import lax
from jax.experimental import pallas as pl
from jax.experimental.pallas import tpu as pltpu
from jax.experimental.pallas import tpu_sc as plsc

BATCH = 16384
EMBED_K = 16
NUM_WORKERS = 32
PAIRS_PER_WORKER = BATCH // NUM_WORKERS
IDX_CHUNK = 128
NUM_CHUNKS = PAIRS_PER_WORKER // IDX_CHUNK


@functools.partial(
    pl.kernel,
    out_type=jax.ShapeDtypeStruct((BATCH,), jnp.float32),
    mesh=plsc.VectorSubcoreMesh(core_axis_name="c", subcore_axis_name="s"),
    compiler_params=pltpu.CompilerParams(use_tc_tiling_on_sc=False),
    scratch_types=[
        pltpu.VMEM((NUM_CHUNKS, IDX_CHUNK), jnp.int32),
        pltpu.VMEM((NUM_CHUNKS, IDX_CHUNK), jnp.int32),
        pltpu.VMEM((PAIRS_PER_WORKER, EMBED_K), jnp.float32),
        pltpu.VMEM((PAIRS_PER_WORKER, EMBED_K), jnp.float32),
        pltpu.VMEM((PAIRS_PER_WORKER,), jnp.float32),
        pltpu.SemaphoreType.DMA,
    ],
)
def _mf_sc_kernel(uidx_hbm, vidx_hbm, w_hbm, h_hbm, out_hbm,
                  uidx_v, vidx_v, u_v, v_v, out_v, sem):
    num_cores = 2
    wid = lax.axis_index("s") * num_cores + lax.axis_index("c")
    base = wid * PAIRS_PER_WORKER

    pltpu.sync_copy(uidx_hbm.at[wid], uidx_v)
    pltpu.sync_copy(vidx_hbm.at[wid], vidx_v)

    copies = []
    for j in range(NUM_CHUNKS):
        dst = pl.ds(j * IDX_CHUNK, IDX_CHUNK)
        copies.append(pltpu.async_copy(w_hbm.at[uidx_v.at[j]], u_v.at[dst], sem))
        copies.append(pltpu.async_copy(h_hbm.at[vidx_v.at[j]], v_v.at[dst], sem))
    for c in copies:
        c.wait()

    lane = lax.iota(jnp.int32, EMBED_K)
    folds = [lane ^ 8, lane ^ 4, lane ^ 2, lane ^ 1]
    masks = [lane % (2 * g) < g for g in (8, 4, 2, 1)]
    bitrev = [0, 8, 4, 12, 2, 10, 6, 14, 1, 9, 5, 13, 3, 11, 7, 15]

    gather_dnums = lax.GatherDimensionNumbers(
        offset_dims=(), collapsed_slice_dims=(0,), start_index_map=(0,))

    def permute(vec, idx):
        return lax.gather(vec, idx[:, None], gather_dnums, (1,),
                          mode=lax.GatherScatterMode.PROMISE_IN_BOUNDS)

    def fold(vec, level):
        return vec + permute(vec, folds[level])

    def block_body(blk, carry):
        regs = []
        for i in bitrev:
            r = blk * EMBED_K + i
            regs.append(u_v[r] * v_v[r])
        for level in range(4):
            nxt = []
            for j in range(0, len(regs), 2):
                a = fold(regs[j], level)
                b = fold(regs[j + 1], level)
                nxt.append(jnp.where(masks[level], a, b))
            regs = nxt
        acc = regs[0]
        sig = 1.0 / (1.0 + jnp.exp(-acc))
        out_v[pl.ds(blk * EMBED_K, EMBED_K)] = sig
        return carry

    lax.fori_loop(0, PAIRS_PER_WORKER // EMBED_K, block_body, 0)

    pltpu.sync_copy(out_v, out_hbm.at[pl.ds(base, PAIRS_PER_WORKER)])


def kernel(x, W, H):
    uidx = x[:, 0].astype(jnp.int32).reshape(NUM_WORKERS, NUM_CHUNKS, IDX_CHUNK)
    vidx = x[:, 1].astype(jnp.int32).reshape(NUM_WORKERS, NUM_CHUNKS, IDX_CHUNK)
    return _mf_sc_kernel(uidx, vidx, W, H)
